# baseline (speedup 1.0000x reference)
import jax
import jax.numpy as jnp
from jax import lax
from jax.experimental import pallas as pl
from jax.experimental.pallas import tpu as pltpu

NUM_SP = 256
TH = 8


def _fused_body(x0_ref, x1_ref, s0_ref, s1_ref, i0_ref, i1_ref, attn_ref,
                Wv_ref, bv_ref, Wp_ref, bp_ref, out_ref, *, W, HD, hd, K, NSP):
    C = HD * hd
    P = K // 2
    WP = W + 2 * P
    HALO = TH + K - 1

    x_loc = jnp.concatenate([x0_ref[0], x1_ref[0]], axis=0)[:HALO]
    sims_loc = jnp.concatenate([s0_ref[0], s1_ref[0]], axis=0)[:HALO]
    sinds_loc = jnp.concatenate([i0_ref[0], i1_ref[0]], axis=0)[:HALO]

    v_loc = jnp.dot(x_loc.reshape(HALO * WP, C), Wv_ref[...],
                    preferred_element_type=jnp.float32) + bv_ref[...]
    v_loc = v_loc.reshape(HALO, WP, C)

    sp_iota = lax.broadcasted_iota(jnp.int32, (HALO, WP, NUM_SP), 2)
    S = jnp.zeros((HALO, WP, NUM_SP), jnp.float32)
    for i in range(NSP):
        onehot = (sinds_loc[:, :, i:i + 1] == sp_iota).astype(jnp.float32)
        S = S + sims_loc[:, :, i:i + 1] * onehot
    S_c = S[P:P + TH, P:P + W, :]

    attn_loc = attn_ref[0].reshape(TH * W, K * K * HD)

    e_row = lax.broadcasted_iota(jnp.int32, (HD, C), 0)
    e_col = lax.broadcasted_iota(jnp.int32, (HD, C), 1) // hd
    E = (e_row == e_col).astype(jnp.float32)

    acc = jnp.zeros((TH, W, C), jnp.float32)
    for dy in range(K):
        v_row = v_loc[dy:dy + TH]
        S_row = S[dy:dy + TH]
        for dx in range(K):
            idx = dy * K + dx
            wgt = jnp.sum(S_c * S_row[:, dx:dx + W, :], axis=-1, keepdims=True)
            a = jnp.dot(attn_loc[:, idx * HD:(idx + 1) * HD], E,
                        preferred_element_type=jnp.float32).reshape(TH, W, C)
            acc = acc + (a * wgt) * v_row[:, dx:dx + W, :]

    out = jnp.dot(acc.reshape(TH * W, C), Wp_ref[...],
                  preferred_element_type=jnp.float32) + bp_ref[...]
    out_ref[0] = out.reshape(TH, W, C)


def kernel(x, attn, sims, sinds, Wv, bv, Wp, bp):
    B, H, W, C = x.shape
    HD = attn.shape[1]
    hd = C // HD
    K = 7
    NSP = sims.shape[-1]
    P = K // 2
    assert H % TH == 0
    NT = H // TH
    HP = (NT + 1) * TH
    WP = W + 2 * P

    pad_hw = ((0, 0), (P, HP - H - P), (P, P), (0, 0))
    x_p = jnp.pad(x, pad_hw)
    sims_p = jnp.pad(sims, pad_hw)
    sinds_p = jnp.pad(sinds, pad_hw)
    attn2 = attn.transpose(0, 2, 3, 4, 1).reshape(B, H, W, K * K * HD)
    bv2 = bv.reshape(1, C)
    bp2 = bp.reshape(1, C)

    import functools
    body = functools.partial(_fused_body, W=W, HD=HD, hd=hd, K=K, NSP=NSP)

    def hm(b, i):
        return (b, i, 0, 0)

    def hm1(b, i):
        return (b, i + 1, 0, 0)

    grid = (B, NT)
    out = pl.pallas_call(
        body,
        grid=grid,
        in_specs=[
            pl.BlockSpec((1, TH, WP, C), hm),
            pl.BlockSpec((1, TH, WP, C), hm1),
            pl.BlockSpec((1, TH, WP, NSP), hm),
            pl.BlockSpec((1, TH, WP, NSP), hm1),
            pl.BlockSpec((1, TH, WP, NSP), hm),
            pl.BlockSpec((1, TH, WP, NSP), hm1),
            pl.BlockSpec((1, TH, W, K * K * HD), hm),
            pl.BlockSpec((C, C), lambda b, i: (0, 0)),
            pl.BlockSpec((1, C), lambda b, i: (0, 0)),
            pl.BlockSpec((C, C), lambda b, i: (0, 0)),
            pl.BlockSpec((1, C), lambda b, i: (0, 0)),
        ],
        out_specs=pl.BlockSpec((1, TH, W, C), hm),
        out_shape=jax.ShapeDtypeStruct((B, H, W, C), jnp.float32),
        compiler_params=pltpu.CompilerParams(
            dimension_semantics=("parallel", "arbitrary"),
            vmem_limit_bytes=96 * 1024 * 1024,
        ),
    )(x_p, x_p, sims_p, sims_p, sinds_p, sinds_p, attn2, Wv, bv2, Wp, bp2)
    return out


# speedup vs baseline: 1.1142x; 1.1142x over previous
import functools
import jax
import jax.numpy as jnp
from jax import lax
from jax.experimental import pallas as pl
from jax.experimental.pallas import tpu as pltpu

NUM_SP = 256
TH = 8


def _fused_body(xm_ref, x0_ref, xp_ref, sm_ref, s0_ref, sp_ref,
                im_ref, i0_ref, ip_ref, attn_ref,
                Wv_ref, bv_ref, Wp_ref, bp_ref, out_ref, *, H, W, HD, hd, K, NSP):
    C = HD * hd
    P = K // 2
    HALO = TH + 2 * P
    i = pl.program_id(1)

    x_loc = jnp.concatenate(
        [xm_ref[0, TH - P:], x0_ref[0], xp_ref[0, :P]], axis=0)
    sims_loc = jnp.concatenate(
        [sm_ref[0, TH - P:], s0_ref[0], sp_ref[0, :P]], axis=0)
    sinds_loc = jnp.concatenate(
        [im_ref[0, TH - P:], i0_ref[0], ip_ref[0, :P]], axis=0)

    glob = TH * i - P + lax.broadcasted_iota(jnp.int32, (HALO, W, 1), 0)
    row_ok = jnp.logical_and(glob >= 0, glob < H)
    sims_loc = jnp.where(row_ok, sims_loc, 0.0)

    v_loc = jnp.dot(x_loc.reshape(HALO * W, C), Wv_ref[...],
                    preferred_element_type=jnp.float32) + bv_ref[...]
    v_loc = v_loc.reshape(HALO, W, C)

    sp_iota = lax.broadcasted_iota(jnp.int32, (HALO, W, NUM_SP), 2)
    S = jnp.zeros((HALO, W, NUM_SP), jnp.float32)
    for q in range(NSP):
        onehot = (sinds_loc[:, :, q:q + 1] == sp_iota).astype(jnp.float32)
        S = S + sims_loc[:, :, q:q + 1] * onehot
    S_c = S[P:P + TH]

    attn_loc = attn_ref[0].reshape(TH * W, K * K * HD)

    e_row = lax.broadcasted_iota(jnp.int32, (HD, C), 0)
    e_col = lax.broadcasted_iota(jnp.int32, (HD, C), 1) // hd
    E = (e_row == e_col).astype(jnp.float32)

    col = lax.broadcasted_iota(jnp.int32, (TH, W, 1), 1)

    acc = jnp.zeros((TH, W, C), jnp.float32)
    for dy in range(K):
        v_row = v_loc[dy:dy + TH]
        S_row = S[dy:dy + TH]
        for dx in range(K):
            idx = dy * K + dx
            sh = dx - P
            if sh == 0:
                S_n, v_n = S_row, v_row
            else:
                S_n = jnp.roll(S_row, -sh, axis=1)
                v_n = jnp.roll(v_row, -sh, axis=1)
            wgt = jnp.sum(S_c * S_n, axis=-1, keepdims=True)
            if sh != 0:
                cmask = jnp.logical_and(col + sh >= 0, col + sh < W)
                wgt = jnp.where(cmask, wgt, 0.0)
            a = jnp.dot(attn_loc[:, idx * HD:(idx + 1) * HD], E,
                        preferred_element_type=jnp.float32).reshape(TH, W, C)
            acc = acc + (a * wgt) * v_n

    out = jnp.dot(acc.reshape(TH * W, C), Wp_ref[...],
                  preferred_element_type=jnp.float32) + bp_ref[...]
    out_ref[0] = out.reshape(TH, W, C)


def kernel(x, attn, sims, sinds, Wv, bv, Wp, bp):
    B, H, W, C = x.shape
    HD = attn.shape[1]
    hd = C // HD
    K = 7
    NSP = sims.shape[-1]
    assert H % TH == 0
    NT = H // TH

    attn2 = attn.transpose(0, 2, 3, 4, 1).reshape(B, H, W, K * K * HD)
    bv2 = bv.reshape(1, C)
    bp2 = bp.reshape(1, C)

    body = functools.partial(_fused_body, H=H, W=W, HD=HD, hd=hd, K=K, NSP=NSP)

    def hm_m(b, i):
        return (b, jnp.maximum(i - 1, 0), 0, 0)

    def hm(b, i):
        return (b, i, 0, 0)

    def hm_p(b, i):
        return (b, jnp.minimum(i + 1, NT - 1), 0, 0)

    def wspec(lastdim):
        return [pl.BlockSpec((1, TH, W, lastdim), m) for m in (hm_m, hm, hm_p)]

    grid = (B, NT)
    out = pl.pallas_call(
        body,
        grid=grid,
        in_specs=(wspec(C) + wspec(NSP) + wspec(NSP) + [
            pl.BlockSpec((1, TH, W, K * K * HD), hm),
            pl.BlockSpec((C, C), lambda b, i: (0, 0)),
            pl.BlockSpec((1, C), lambda b, i: (0, 0)),
            pl.BlockSpec((C, C), lambda b, i: (0, 0)),
            pl.BlockSpec((1, C), lambda b, i: (0, 0)),
        ]),
        out_specs=pl.BlockSpec((1, TH, W, C), hm),
        out_shape=jax.ShapeDtypeStruct((B, H, W, C), jnp.float32),
        compiler_params=pltpu.CompilerParams(
            dimension_semantics=("parallel", "arbitrary"),
            vmem_limit_bytes=100 * 1024 * 1024,
        ),
    )(x, x, x, sims, sims, sims, sinds, sinds, sinds, attn2, Wv, bv2, Wp, bp2)
    return out


# speedup vs baseline: 1.1302x; 1.0143x over previous
import functools
import jax
import jax.numpy as jnp
from jax import lax
from jax.experimental import pallas as pl
from jax.experimental.pallas import tpu as pltpu

NUM_SP = 256
TH = 8


def _fused_body(xm_ref, x0_ref, xp_ref, sm_ref, s0_ref, sp_ref,
                im_ref, i0_ref, ip_ref, attn_ref,
                Wv_ref, bv_ref, Wp_ref, bp_ref, out_ref, *, H, W, HD, hd, K, NSP):
    C = HD * hd
    P = K // 2
    HALO = TH + 2 * P
    i = pl.program_id(1)

    x_loc = jnp.concatenate(
        [xm_ref[0, TH - P:], x0_ref[0], xp_ref[0, :P]], axis=0)
    sims_loc = jnp.concatenate(
        [sm_ref[0, TH - P:], s0_ref[0], sp_ref[0, :P]], axis=0)
    sinds_loc = jnp.concatenate(
        [im_ref[0, TH - P:], i0_ref[0], ip_ref[0, :P]], axis=0)

    glob = TH * i - P + lax.broadcasted_iota(jnp.int32, (HALO, W, 1), 0)
    row_ok = jnp.logical_and(glob >= 0, glob < H)
    sims_loc = jnp.where(row_ok, sims_loc, 0.0)

    v_loc = jnp.dot(x_loc.reshape(HALO * W, C), Wv_ref[...],
                    preferred_element_type=jnp.float32) + bv_ref[...]
    v_loc = v_loc.reshape(HALO, W, C)

    sp_iota = lax.broadcasted_iota(jnp.int32, (HALO, W, NUM_SP), 2)
    S = jnp.zeros((HALO, W, NUM_SP), jnp.float32)
    for q in range(NSP):
        onehot = (sinds_loc[:, :, q:q + 1] == sp_iota).astype(jnp.float32)
        S = S + sims_loc[:, :, q:q + 1] * onehot
    S_c = S[P:P + TH]

    attn_loc = attn_ref[0].reshape(TH * W, K * K * HD)

    e_row = lax.broadcasted_iota(jnp.int32, (HD, C), 0)
    e_col = lax.broadcasted_iota(jnp.int32, (HD, C), 1) // hd
    E = (e_row == e_col).astype(jnp.float32)

    col = lax.broadcasted_iota(jnp.int32, (TH, W, 1), 1)
    ones_col = jnp.ones((NUM_SP, 1), jnp.float32)

    acc = jnp.zeros((TH, W, C), jnp.float32)
    for dx in range(K):
        sh = dx - P
        if sh == 0:
            S_sh, v_sh = S, v_loc
            cmask = None
        else:
            S_sh = jnp.roll(S, -sh, axis=1)
            v_sh = jnp.roll(v_loc, -sh, axis=1)
            cmask = jnp.logical_and(col + sh >= 0, col + sh < W)
        for dy in range(K):
            idx = dy * K + dx
            S_n = S_sh[dy:dy + TH]
            v_n = v_sh[dy:dy + TH]
            wgt = jnp.dot((S_c * S_n).reshape(TH * W, NUM_SP), ones_col,
                          preferred_element_type=jnp.float32).reshape(TH, W, 1)
            if cmask is not None:
                wgt = jnp.where(cmask, wgt, 0.0)
            a = jnp.dot(attn_loc[:, idx * HD:(idx + 1) * HD], E,
                        preferred_element_type=jnp.float32).reshape(TH, W, C)
            acc = acc + (a * wgt) * v_n

    out = jnp.dot(acc.reshape(TH * W, C), Wp_ref[...],
                  preferred_element_type=jnp.float32) + bp_ref[...]
    out_ref[0] = out.reshape(TH, W, C)


def kernel(x, attn, sims, sinds, Wv, bv, Wp, bp):
    B, H, W, C = x.shape
    HD = attn.shape[1]
    hd = C // HD
    K = 7
    NSP = sims.shape[-1]
    assert H % TH == 0
    NT = H // TH

    attn2 = attn.transpose(0, 2, 3, 4, 1).reshape(B, H, W, K * K * HD)
    bv2 = bv.reshape(1, C)
    bp2 = bp.reshape(1, C)

    body = functools.partial(_fused_body, H=H, W=W, HD=HD, hd=hd, K=K, NSP=NSP)

    def hm_m(b, i):
        return (b, jnp.maximum(i - 1, 0), 0, 0)

    def hm(b, i):
        return (b, i, 0, 0)

    def hm_p(b, i):
        return (b, jnp.minimum(i + 1, NT - 1), 0, 0)

    def wspec(lastdim):
        return [pl.BlockSpec((1, TH, W, lastdim), m) for m in (hm_m, hm, hm_p)]

    grid = (B, NT)
    out = pl.pallas_call(
        body,
        grid=grid,
        in_specs=(wspec(C) + wspec(NSP) + wspec(NSP) + [
            pl.BlockSpec((1, TH, W, K * K * HD), hm),
            pl.BlockSpec((C, C), lambda b, i: (0, 0)),
            pl.BlockSpec((1, C), lambda b, i: (0, 0)),
            pl.BlockSpec((C, C), lambda b, i: (0, 0)),
            pl.BlockSpec((1, C), lambda b, i: (0, 0)),
        ]),
        out_specs=pl.BlockSpec((1, TH, W, C), hm),
        out_shape=jax.ShapeDtypeStruct((B, H, W, C), jnp.float32),
        compiler_params=pltpu.CompilerParams(
            dimension_semantics=("parallel", "arbitrary"),
            vmem_limit_bytes=100 * 1024 * 1024,
        ),
    )(x, x, x, sims, sims, sims, sinds, sinds, sinds, attn2, Wv, bv2, Wp, bp2)
    return out


# speedup vs baseline: 1.1861x; 1.0494x over previous
import functools
import jax
import jax.numpy as jnp
from jax import lax
from jax.experimental import pallas as pl
from jax.experimental.pallas import tpu as pltpu

NUM_SP = 256
TH = 8


def _fused_body(xm_ref, x0_ref, xp_ref, sm_ref, s0_ref, sp_ref,
                im_ref, i0_ref, ip_ref, attn_ref,
                Wv_ref, bv_ref, Wp_ref, bp_ref, out_ref, *, H, W, HD, hd, K, NSP):
    C = HD * hd
    P = K // 2
    HALO = TH + 2 * P
    i = pl.program_id(1)

    x_loc = jnp.concatenate(
        [xm_ref[0, TH - P:], x0_ref[0], xp_ref[0, :P]], axis=0)
    sims_loc = jnp.concatenate(
        [sm_ref[0, TH - P:], s0_ref[0], sp_ref[0, :P]], axis=0)
    sinds_loc = jnp.concatenate(
        [im_ref[0, TH - P:], i0_ref[0], ip_ref[0, :P]], axis=0)

    glob = TH * i - P + lax.broadcasted_iota(jnp.int32, (HALO, W, 1), 0)
    row_ok = jnp.logical_and(glob >= 0, glob < H)
    sims_loc = jnp.where(row_ok, sims_loc, 0.0)

    v_loc = jnp.dot(x_loc.reshape(HALO * W, C), Wv_ref[...],
                    preferred_element_type=jnp.float32) + bv_ref[...]
    v_loc = v_loc.reshape(HALO, W, C)

    sp_iota = lax.broadcasted_iota(jnp.int32, (HALO, W, NUM_SP), 2)
    S = jnp.zeros((HALO, W, NUM_SP), jnp.float32)
    for q in range(NSP):
        onehot = (sinds_loc[:, :, q:q + 1] == sp_iota).astype(jnp.float32)
        S = S + sims_loc[:, :, q:q + 1] * onehot
    S_c = S[P:P + TH]

    attn_loc = attn_ref[0].reshape(TH * W, K * K * HD)

    e_row = lax.broadcasted_iota(jnp.int32, (HD, C), 0)
    e_col = lax.broadcasted_iota(jnp.int32, (HD, C), 1) // hd
    E = (e_row == e_col).astype(jnp.float32)

    col = lax.broadcasted_iota(jnp.int32, (HALO, W, 1), 1)
    ones_bc = jnp.ones((NUM_SP, C), jnp.float32)

    acc = jnp.zeros((TH, W, C), jnp.float32)
    for dx in range(K):
        sh = dx - P
        if sh == 0:
            S_sh, v_sh = S, v_loc
        else:
            cmask = jnp.logical_and(col + sh >= 0, col + sh < W)
            S_sh = jnp.where(cmask, jnp.roll(S, -sh, axis=1), 0.0)
            v_sh = jnp.roll(v_loc, -sh, axis=1)
        for dy in range(K):
            idx = dy * K + dx
            S_n = S_sh[dy:dy + TH]
            v_n = v_sh[dy:dy + TH]
            wgt_b = jnp.dot((S_c * S_n).reshape(TH * W, NUM_SP), ones_bc,
                            preferred_element_type=jnp.float32
                            ).reshape(TH, W, C)
            a = jnp.dot(attn_loc[:, idx * HD:(idx + 1) * HD], E,
                        preferred_element_type=jnp.float32).reshape(TH, W, C)
            acc = acc + (a * wgt_b) * v_n

    out = jnp.dot(acc.reshape(TH * W, C), Wp_ref[...],
                  preferred_element_type=jnp.float32) + bp_ref[...]
    out_ref[0] = out.reshape(TH, W, C)


def kernel(x, attn, sims, sinds, Wv, bv, Wp, bp):
    B, H, W, C = x.shape
    HD = attn.shape[1]
    hd = C // HD
    K = 7
    NSP = sims.shape[-1]
    assert H % TH == 0
    NT = H // TH

    attn2 = attn.transpose(0, 2, 3, 4, 1).reshape(B, H, W, K * K * HD)
    bv2 = bv.reshape(1, C)
    bp2 = bp.reshape(1, C)

    body = functools.partial(_fused_body, H=H, W=W, HD=HD, hd=hd, K=K, NSP=NSP)

    def hm_m(b, i):
        return (b, jnp.maximum(i - 1, 0), 0, 0)

    def hm(b, i):
        return (b, i, 0, 0)

    def hm_p(b, i):
        return (b, jnp.minimum(i + 1, NT - 1), 0, 0)

    def wspec(lastdim):
        return [pl.BlockSpec((1, TH, W, lastdim), m) for m in (hm_m, hm, hm_p)]

    grid = (B, NT)
    out = pl.pallas_call(
        body,
        grid=grid,
        in_specs=(wspec(C) + wspec(NSP) + wspec(NSP) + [
            pl.BlockSpec((1, TH, W, K * K * HD), hm),
            pl.BlockSpec((C, C), lambda b, i: (0, 0)),
            pl.BlockSpec((1, C), lambda b, i: (0, 0)),
            pl.BlockSpec((C, C), lambda b, i: (0, 0)),
            pl.BlockSpec((1, C), lambda b, i: (0, 0)),
        ]),
        out_specs=pl.BlockSpec((1, TH, W, C), hm),
        out_shape=jax.ShapeDtypeStruct((B, H, W, C), jnp.float32),
        compiler_params=pltpu.CompilerParams(
            dimension_semantics=("parallel", "arbitrary"),
            vmem_limit_bytes=100 * 1024 * 1024,
        ),
    )(x, x, x, sims, sims, sims, sinds, sinds, sinds, attn2, Wv, bv2, Wp, bp2)
    return out


# speedup vs baseline: 1.4878x; 1.2544x over previous
import functools
import jax
import jax.numpy as jnp
from jax import lax
from jax.experimental import pallas as pl
from jax.experimental.pallas import tpu as pltpu

NUM_SP = 256
TH = 8


def _fused_body(xm_ref, x0_ref, xp_ref, sm_ref, s0_ref, sp_ref,
                im_ref, i0_ref, ip_ref, attn_ref,
                Wv_ref, bv_ref, Wp_ref, bp_ref, out_ref, *, H, W, HD, hd, K, NSP):
    C = HD * hd
    P = K // 2
    HALO = TH + 2 * P
    i = pl.program_id(1)

    x_loc = jnp.concatenate(
        [xm_ref[0, TH - P:], x0_ref[0], xp_ref[0, :P]], axis=0)
    sims_loc = jnp.concatenate(
        [sm_ref[0, TH - P:], s0_ref[0], sp_ref[0, :P]], axis=0)
    sinds_loc = jnp.concatenate(
        [im_ref[0, TH - P:], i0_ref[0], ip_ref[0, :P]], axis=0)

    glob = TH * i - P + lax.broadcasted_iota(jnp.int32, (HALO, W, 1), 0)
    row_ok = jnp.logical_and(glob >= 0, glob < H)
    sims_loc = jnp.where(row_ok, sims_loc, 0.0)

    v_loc = jnp.dot(x_loc.reshape(HALO * W, C), Wv_ref[...],
                    preferred_element_type=jnp.float32) + bv_ref[...]
    v_loc = v_loc.reshape(HALO, W, C)

    sp_iota = lax.broadcasted_iota(jnp.int32, (HALO, W, NUM_SP), 2)
    S = jnp.zeros((HALO, W, NUM_SP), jnp.float32)
    for q in range(NSP):
        onehot = (sinds_loc[:, :, q:q + 1] == sp_iota).astype(jnp.float32)
        S = S + sims_loc[:, :, q:q + 1] * onehot
    S_bf = S.astype(jnp.bfloat16)
    S_c = S_bf[P:P + TH]
    S_pad = jnp.concatenate(
        [jnp.zeros((HALO, P, NUM_SP), jnp.bfloat16), S_bf,
         jnp.zeros((HALO, P, NUM_SP), jnp.bfloat16)], axis=1)

    attn_loc = attn_ref[0].reshape(TH * W, K * K * HD)

    e_row = lax.broadcasted_iota(jnp.int32, (HD, C), 0)
    e_col = lax.broadcasted_iota(jnp.int32, (HD, C), 1) // hd
    E = (e_row == e_col).astype(jnp.float32)

    ones_bc = jnp.ones((NUM_SP, C), jnp.bfloat16)

    acc = jnp.zeros((TH, W, C), jnp.float32)
    for dx in range(K):
        sh = dx - P
        S_sh = S_pad[:, dx:dx + W]
        v_sh = v_loc if sh == 0 else jnp.roll(v_loc, -sh, axis=1)
        for dy in range(K):
            idx = dy * K + dx
            S_n = S_sh[dy:dy + TH]
            v_n = v_sh[dy:dy + TH]
            wgt_b = jnp.dot((S_c * S_n).reshape(TH * W, NUM_SP), ones_bc,
                            preferred_element_type=jnp.float32
                            ).reshape(TH, W, C)
            a = jnp.dot(attn_loc[:, idx * HD:(idx + 1) * HD], E,
                        preferred_element_type=jnp.float32).reshape(TH, W, C)
            acc = acc + (a * wgt_b) * v_n

    out = jnp.dot(acc.reshape(TH * W, C), Wp_ref[...],
                  preferred_element_type=jnp.float32) + bp_ref[...]
    out_ref[0] = out.reshape(TH, W, C)


def kernel(x, attn, sims, sinds, Wv, bv, Wp, bp):
    B, H, W, C = x.shape
    HD = attn.shape[1]
    hd = C // HD
    K = 7
    NSP = sims.shape[-1]
    assert H % TH == 0
    NT = H // TH

    attn2 = attn.transpose(0, 2, 3, 4, 1).reshape(B, H, W, K * K * HD)
    bv2 = bv.reshape(1, C)
    bp2 = bp.reshape(1, C)

    body = functools.partial(_fused_body, H=H, W=W, HD=HD, hd=hd, K=K, NSP=NSP)

    def hm_m(b, i):
        return (b, jnp.maximum(i - 1, 0), 0, 0)

    def hm(b, i):
        return (b, i, 0, 0)

    def hm_p(b, i):
        return (b, jnp.minimum(i + 1, NT - 1), 0, 0)

    def wspec(lastdim):
        return [pl.BlockSpec((1, TH, W, lastdim), m) for m in (hm_m, hm, hm_p)]

    grid = (B, NT)
    out = pl.pallas_call(
        body,
        grid=grid,
        in_specs=(wspec(C) + wspec(NSP) + wspec(NSP) + [
            pl.BlockSpec((1, TH, W, K * K * HD), hm),
            pl.BlockSpec((C, C), lambda b, i: (0, 0)),
            pl.BlockSpec((1, C), lambda b, i: (0, 0)),
            pl.BlockSpec((C, C), lambda b, i: (0, 0)),
            pl.BlockSpec((1, C), lambda b, i: (0, 0)),
        ]),
        out_specs=pl.BlockSpec((1, TH, W, C), hm),
        out_shape=jax.ShapeDtypeStruct((B, H, W, C), jnp.float32),
        compiler_params=pltpu.CompilerParams(
            dimension_semantics=("parallel", "arbitrary"),
            vmem_limit_bytes=100 * 1024 * 1024,
        ),
    )(x, x, x, sims, sims, sims, sinds, sinds, sinds, attn2, Wv, bv2, Wp, bp2)
    return out


# speedup vs baseline: 1.5338x; 1.0309x over previous
import functools
import jax
import jax.numpy as jnp
from jax import lax
from jax.experimental import pallas as pl
from jax.experimental.pallas import tpu as pltpu

NUM_SP = 256
TH = 8


def _fused_body(xm_ref, x0_ref, xp_ref, sm_ref, s0_ref, sp_ref,
                im_ref, i0_ref, ip_ref, attn_ref,
                Wv_ref, bv_ref, Wp_ref, bp_ref, out_ref, *, H, W, HD, hd, K, NSP):
    C = HD * hd
    P = K // 2
    HALO = TH + 2 * P
    i = pl.program_id(1)

    x_loc = jnp.concatenate(
        [xm_ref[0, TH - P:], x0_ref[0], xp_ref[0, :P]], axis=0)
    sims_loc = jnp.concatenate(
        [sm_ref[0, TH - P:], s0_ref[0], sp_ref[0, :P]], axis=0)
    sinds_loc = jnp.concatenate(
        [im_ref[0, TH - P:], i0_ref[0], ip_ref[0, :P]], axis=0)

    glob = TH * i - P + lax.broadcasted_iota(jnp.int32, (HALO, W, 1), 0)
    row_ok = jnp.logical_and(glob >= 0, glob < H)
    sims_loc = jnp.where(row_ok, sims_loc, 0.0)

    v_loc = jnp.dot(x_loc.reshape(HALO * W, C), Wv_ref[...],
                    preferred_element_type=jnp.float32) + bv_ref[...]
    v_loc = v_loc.reshape(HALO, W, C)

    sp_iota = lax.broadcasted_iota(jnp.int32, (HALO, W, NUM_SP), 2)
    S = jnp.zeros((HALO, W, NUM_SP), jnp.float32)
    for q in range(NSP):
        onehot = (sinds_loc[:, :, q:q + 1] == sp_iota).astype(jnp.float32)
        S = S + sims_loc[:, :, q:q + 1] * onehot
    S_bf = S.astype(jnp.bfloat16)
    S_c = S_bf[P:P + TH]
    S_pad = jnp.concatenate(
        [jnp.zeros((HALO, P, NUM_SP), jnp.bfloat16), S_bf,
         jnp.zeros((HALO, P, NUM_SP), jnp.bfloat16)], axis=1)

    attn_loc = attn_ref[0].reshape(TH * W, K * K * HD)

    e_row = lax.broadcasted_iota(jnp.int32, (HD, C), 0)
    e_col = lax.broadcasted_iota(jnp.int32, (HD, C), 1) // hd
    E = (e_row == e_col).astype(jnp.bfloat16)

    ones_bc = jnp.ones((NUM_SP, C), jnp.bfloat16)

    acc = jnp.zeros((TH, W, C), jnp.float32)
    for dx in range(K):
        sh = dx - P
        S_sh = S_pad[:, dx:dx + W]
        v_sh = v_loc if sh == 0 else jnp.roll(v_loc, -sh, axis=1)
        for dy in range(K):
            idx = dy * K + dx
            S_n = S_sh[dy:dy + TH]
            v_n = v_sh[dy:dy + TH]
            wgt_b = jnp.dot((S_c * S_n).reshape(TH * W, NUM_SP), ones_bc,
                            preferred_element_type=jnp.float32
                            ).reshape(TH, W, C)
            a = jnp.dot(attn_loc[:, idx * HD:(idx + 1) * HD], E,
                        preferred_element_type=jnp.float32).reshape(TH, W, C)
            acc = acc + (a * wgt_b) * v_n

    out = jnp.dot(acc.reshape(TH * W, C), Wp_ref[...],
                  preferred_element_type=jnp.float32) + bp_ref[...]
    out_ref[0] = out.reshape(TH, W, C)


def kernel(x, attn, sims, sinds, Wv, bv, Wp, bp):
    B, H, W, C = x.shape
    HD = attn.shape[1]
    hd = C // HD
    K = 7
    NSP = sims.shape[-1]
    assert H % TH == 0
    NT = H // TH

    attn2 = attn.transpose(0, 2, 3, 4, 1).reshape(B, H, W, K * K * HD)
    attn2 = attn2.astype(jnp.bfloat16)
    bv2 = bv.reshape(1, C)
    bp2 = bp.reshape(1, C)

    body = functools.partial(_fused_body, H=H, W=W, HD=HD, hd=hd, K=K, NSP=NSP)

    def hm_m(b, i):
        return (b, jnp.maximum(i - 1, 0), 0, 0)

    def hm(b, i):
        return (b, i, 0, 0)

    def hm_p(b, i):
        return (b, jnp.minimum(i + 1, NT - 1), 0, 0)

    def wspec(lastdim):
        return [pl.BlockSpec((1, TH, W, lastdim), m) for m in (hm_m, hm, hm_p)]

    grid = (B, NT)
    out = pl.pallas_call(
        body,
        grid=grid,
        in_specs=(wspec(C) + wspec(NSP) + wspec(NSP) + [
            pl.BlockSpec((1, TH, W, K * K * HD), hm),
            pl.BlockSpec((C, C), lambda b, i: (0, 0)),
            pl.BlockSpec((1, C), lambda b, i: (0, 0)),
            pl.BlockSpec((C, C), lambda b, i: (0, 0)),
            pl.BlockSpec((1, C), lambda b, i: (0, 0)),
        ]),
        out_specs=pl.BlockSpec((1, TH, W, C), hm),
        out_shape=jax.ShapeDtypeStruct((B, H, W, C), jnp.float32),
        compiler_params=pltpu.CompilerParams(
            dimension_semantics=("parallel", "arbitrary"),
            vmem_limit_bytes=100 * 1024 * 1024,
        ),
    )(x, x, x, sims, sims, sims, sinds, sinds, sinds, attn2, Wv, bv2, Wp, bp2)
    return out
